# baseline (speedup 1.0000x reference)
import functools

import jax
import jax.numpy as jnp
from jax import lax
from jax.experimental import pallas as pl
from jax.experimental.pallas import tpu as pltpu
from jax.experimental.pallas import tpu_sc as plsc

B, L, D = 4096, 50, 64
V_ITEMS, V_RATINGS = 100000, 5
BL = B * L
BL2 = BL // 2
B2 = B // 2

NC, NS = 2, 16
NW = NC * NS
CHUNK = 320
NBUF = 4
NSPLIT = 2
BLS = BL // NSPLIT
B_PER_W = BLS // NW
NCHUNK = B_PER_W // CHUNK


def _sc_gather(table, idx_flat):
    mesh = plsc.VectorSubcoreMesh(core_axis_name="c", subcore_axis_name="s")

    @functools.partial(
        pl.kernel,
        mesh=mesh,
        compiler_params=pltpu.CompilerParams(use_tc_tiling_on_sc=False),
        out_type=jax.ShapeDtypeStruct((BLS, D), jnp.float32),
        scratch_types=(
            [pltpu.VMEM((B_PER_W,), jnp.int32)]
            + [pltpu.VMEM((CHUNK, D), jnp.float32) for _ in range(NBUF)]
            + [pltpu.SemaphoreType.DMA for _ in range(2 * NBUF)]
        ),
    )
    def k(table_hbm, idx_hbm, out_hbm, idx_v, *bufs_sems):
        bufs = bufs_sems[:NBUF]
        sems = bufs_sems[NBUF:2 * NBUF]
        osems = bufs_sems[2 * NBUF:]
        wid = lax.axis_index("s") * NC + lax.axis_index("c")
        base = wid * B_PER_W
        pltpu.sync_copy(idx_hbm.at[pl.ds(base, B_PER_W)], idx_v)

        def gather(j):
            return pltpu.async_copy(
                table_hbm.at[idx_v.at[pl.ds(j * CHUNK, CHUNK)]],
                bufs[j % NBUF], sems[j % NBUF])

        copies = [None] * NCHUNK
        ocopies = [None] * NCHUNK
        for j in range(NBUF - 1):
            copies[j] = gather(j)
        for j in range(NCHUNK):
            jn = j + NBUF - 1
            if jn < NCHUNK:
                if j >= 1:
                    ocopies[j - 1].wait()
                copies[jn] = gather(jn)
            copies[j].wait()
            ocopies[j] = pltpu.async_copy(
                bufs[j % NBUF], out_hbm.at[pl.ds(base + j * CHUNK, CHUNK)],
                osems[j % NBUF])
        for j in range(max(0, NCHUNK - NBUF), NCHUNK):
            if ocopies[j] is not None:
                ocopies[j].wait()

    return k(table, idx_flat)


KL = 5
NSTEP = (L // NSPLIT) // KL


def _tc_mlp(g2, r16, prev, r2e_pad, w1a2, w1b, b1r, w2d, b2d, do_scale):
    def body(g_ref, r_ref, prev_ref, r2e_ref, w1a_ref, w1b_ref, b1_ref,
             w2_ref, b2_ref, out_ref):
        li = pl.program_id(0)
        rtab = jnp.dot(r2e_ref[...], w1b_ref[...],
                       preferred_element_type=jnp.float32) + b1_ref[...]
        z8 = jnp.zeros((8, D), jnp.float32)
        rtab16 = jnp.concatenate(
            [jnp.concatenate([rtab, z8], axis=1),
             jnp.concatenate([z8, rtab], axis=1)], axis=0)
        ohT = (r_ref[...] == lax.broadcasted_iota(
            jnp.int32, (16, KL * B2), 0)).astype(jnp.float32)
        radd = lax.dot_general(ohT, rtab16, (((0,), (0,)), ((), ())),
                               preferred_element_type=jnp.float32)
        h1 = jnp.maximum(
            jnp.dot(g_ref[...], w1a_ref[...],
                    preferred_element_type=jnp.float32) + radd, 0.0)
        h2 = jnp.maximum(
            jnp.dot(h1, w2_ref[...],
                    preferred_element_type=jnp.float32) + b2_ref[...], 0.0)
        part = h2[0:B2, :]
        for kk in range(1, KL):
            part = part + h2[kk * B2:(kk + 1) * B2, :]

        @pl.when(li == 0)
        def _init():
            out_ref[...] = prev_ref[...]

        out_ref[...] += part

        if do_scale:
            @pl.when(li == NSTEP - 1)
            def _scale():
                out_ref[...] *= (1.0 / L)

    return pl.pallas_call(
        body,
        grid=(NSTEP,),
        in_specs=[
            pl.BlockSpec((KL * B2, 2 * D), lambda l: (l, 0)),
            pl.BlockSpec((16, KL * B2), lambda l: (0, l)),
            pl.BlockSpec((B2, 2 * D), lambda l: (0, 0)),
            pl.BlockSpec((8, D), lambda l: (0, 0)),
            pl.BlockSpec((2 * D, 2 * D), lambda l: (0, 0)),
            pl.BlockSpec((D, D), lambda l: (0, 0)),
            pl.BlockSpec((1, D), lambda l: (0, 0)),
            pl.BlockSpec((2 * D, 2 * D), lambda l: (0, 0)),
            pl.BlockSpec((1, 2 * D), lambda l: (0, 0)),
        ],
        out_specs=pl.BlockSpec((B2, 2 * D), lambda l: (0, 0)),
        out_shape=jax.ShapeDtypeStruct((B2, 2 * D), jnp.float32),
        compiler_params=pltpu.CompilerParams(
            dimension_semantics=("arbitrary",)),
    )(g2, r16, prev, r2e_pad, w1a2, w1b, b1r, w2d, b2d)


def _blockdiag2(w):
    z = jnp.zeros_like(w)
    return jnp.concatenate(
        [jnp.concatenate([w, z], axis=1),
         jnp.concatenate([z, w], axis=1)], axis=0)


def kernel(history_uv, history_r, v2e_w, r2e_w, W1, b1, W2, b2):
    idx_flat = history_uv.T.reshape(-1).astype(jnp.int32)
    rp = history_r.T.reshape(BL2, 2).astype(jnp.int32)
    r16 = jnp.concatenate(
        [jnp.broadcast_to(rp[None, :, 0], (8, BL2)),
         jnp.broadcast_to(rp[None, :, 1], (8, BL2)) + 8], axis=0)
    w1a2 = _blockdiag2(W1[:, :D].T)
    w1bT = W1[:, D:].T
    w2d = _blockdiag2(W2.T)
    r2e_pad = jnp.concatenate(
        [r2e_w[4:5], r2e_w[:4], jnp.zeros((3, D), jnp.float32)], axis=0)
    b1r = b1.reshape(1, D)
    b2d = jnp.concatenate([b2, b2]).reshape(1, 2 * D)
    BLS2 = BLS // 2
    acc = jnp.zeros((B2, 2 * D), jnp.float32)
    for s in range(NSPLIT):
        g2 = _sc_gather(
            v2e_w, lax.dynamic_slice_in_dim(idx_flat, s * BLS, BLS)
        ).reshape(BLS2, 2 * D)
        r16s = lax.dynamic_slice_in_dim(r16, s * BLS2, BLS2, axis=1)
        acc = _tc_mlp(g2, r16s, acc, r2e_pad, w1a2, w1bT, b1r, w2d, b2d,
                      do_scale=(s == NSPLIT - 1))
    return acc.reshape(B, D)


# speedup vs baseline: 1.0191x; 1.0191x over previous
import functools

import jax
import jax.numpy as jnp
from jax import lax
from jax.experimental import pallas as pl
from jax.experimental.pallas import tpu as pltpu
from jax.experimental.pallas import tpu_sc as plsc

B, L, D = 4096, 50, 64
V_ITEMS, V_RATINGS = 100000, 5
BL = B * L
BL2 = BL // 2
B2 = B // 2

NC, NS = 2, 16
NW = NC * NS
CHUNK = 320
NBUF = 4
NSPLIT = 1
BLS = BL // NSPLIT
B_PER_W = BLS // NW
NCHUNK = B_PER_W // CHUNK


def _sc_gather(table, idx_flat):
    mesh = plsc.VectorSubcoreMesh(core_axis_name="c", subcore_axis_name="s")

    @functools.partial(
        pl.kernel,
        mesh=mesh,
        compiler_params=pltpu.CompilerParams(use_tc_tiling_on_sc=False),
        out_type=jax.ShapeDtypeStruct((BLS, D), jnp.float32),
        scratch_types=(
            [pltpu.VMEM((B_PER_W,), jnp.int32)]
            + [pltpu.VMEM((CHUNK, D), jnp.float32) for _ in range(NBUF)]
            + [pltpu.SemaphoreType.DMA for _ in range(2 * NBUF)]
        ),
    )
    def k(table_hbm, idx_hbm, out_hbm, idx_v, *bufs_sems):
        bufs = bufs_sems[:NBUF]
        sems = bufs_sems[NBUF:2 * NBUF]
        osems = bufs_sems[2 * NBUF:]
        wid = lax.axis_index("s") * NC + lax.axis_index("c")
        base = wid * B_PER_W
        pltpu.sync_copy(idx_hbm.at[pl.ds(base, B_PER_W)], idx_v)

        def gather(j):
            return pltpu.async_copy(
                table_hbm.at[idx_v.at[pl.ds(j * CHUNK, CHUNK)]],
                bufs[j % NBUF], sems[j % NBUF])

        copies = [None] * NCHUNK
        ocopies = [None] * NCHUNK
        for j in range(NBUF - 1):
            copies[j] = gather(j)
        for j in range(NCHUNK):
            jn = j + NBUF - 1
            if jn < NCHUNK:
                if j >= 1:
                    ocopies[j - 1].wait()
                copies[jn] = gather(jn)
            copies[j].wait()
            ocopies[j] = pltpu.async_copy(
                bufs[j % NBUF], out_hbm.at[pl.ds(base + j * CHUNK, CHUNK)],
                osems[j % NBUF])
        for j in range(max(0, NCHUNK - NBUF), NCHUNK):
            if ocopies[j] is not None:
                ocopies[j].wait()

    return k(table, idx_flat)


KL = 10
NSTEP = (L // NSPLIT) // KL


def _tc_mlp(g2, r16, prev, r2e_pad, w1a2, w1b, b1r, w2d, b2d, do_scale):
    def body(g_ref, r_ref, prev_ref, r2e_ref, w1a_ref, w1b_ref, b1_ref,
             w2_ref, b2_ref, out_ref):
        li = pl.program_id(0)
        rtab = jnp.dot(r2e_ref[...], w1b_ref[...],
                       preferred_element_type=jnp.float32) + b1_ref[...]
        z8 = jnp.zeros((8, D), jnp.float32)
        rtab16 = jnp.concatenate(
            [jnp.concatenate([rtab, z8], axis=1),
             jnp.concatenate([z8, rtab], axis=1)], axis=0)
        ohT = (r_ref[...] == lax.broadcasted_iota(
            jnp.int32, (16, KL * B2), 0)).astype(jnp.float32)
        radd = lax.dot_general(ohT, rtab16, (((0,), (0,)), ((), ())),
                               preferred_element_type=jnp.float32)
        h1 = jnp.maximum(
            jnp.dot(g_ref[...], w1a_ref[...],
                    preferred_element_type=jnp.float32) + radd, 0.0)
        h2 = jnp.maximum(
            jnp.dot(h1, w2_ref[...],
                    preferred_element_type=jnp.float32) + b2_ref[...], 0.0)
        part = h2[0:B2, :]
        for kk in range(1, KL):
            part = part + h2[kk * B2:(kk + 1) * B2, :]

        @pl.when(li == 0)
        def _init():
            out_ref[...] = prev_ref[...]

        out_ref[...] += part

        if do_scale:
            @pl.when(li == NSTEP - 1)
            def _scale():
                out_ref[...] *= (1.0 / L)

    return pl.pallas_call(
        body,
        grid=(NSTEP,),
        in_specs=[
            pl.BlockSpec((KL * B2, 2 * D), lambda l: (l, 0)),
            pl.BlockSpec((16, KL * B2), lambda l: (0, l)),
            pl.BlockSpec((B2, 2 * D), lambda l: (0, 0)),
            pl.BlockSpec((8, D), lambda l: (0, 0)),
            pl.BlockSpec((2 * D, 2 * D), lambda l: (0, 0)),
            pl.BlockSpec((D, D), lambda l: (0, 0)),
            pl.BlockSpec((1, D), lambda l: (0, 0)),
            pl.BlockSpec((2 * D, 2 * D), lambda l: (0, 0)),
            pl.BlockSpec((1, 2 * D), lambda l: (0, 0)),
        ],
        out_specs=pl.BlockSpec((B2, 2 * D), lambda l: (0, 0)),
        out_shape=jax.ShapeDtypeStruct((B2, 2 * D), jnp.float32),
        compiler_params=pltpu.CompilerParams(
            dimension_semantics=("arbitrary",)),
    )(g2, r16, prev, r2e_pad, w1a2, w1b, b1r, w2d, b2d)


def _blockdiag2(w):
    z = jnp.zeros_like(w)
    return jnp.concatenate(
        [jnp.concatenate([w, z], axis=1),
         jnp.concatenate([z, w], axis=1)], axis=0)


def kernel(history_uv, history_r, v2e_w, r2e_w, W1, b1, W2, b2):
    idx_flat = history_uv.T.reshape(-1).astype(jnp.int32)
    rp = history_r.T.reshape(BL2, 2).astype(jnp.int32)
    r16 = jnp.concatenate(
        [jnp.broadcast_to(rp[None, :, 0], (8, BL2)),
         jnp.broadcast_to(rp[None, :, 1], (8, BL2)) + 8], axis=0)
    w1a2 = _blockdiag2(W1[:, :D].T)
    w1bT = W1[:, D:].T
    w2d = _blockdiag2(W2.T)
    r2e_pad = jnp.concatenate(
        [r2e_w[4:5], r2e_w[:4], jnp.zeros((3, D), jnp.float32)], axis=0)
    b1r = b1.reshape(1, D)
    b2d = jnp.concatenate([b2, b2]).reshape(1, 2 * D)
    BLS2 = BLS // 2
    acc = jnp.zeros((B2, 2 * D), jnp.float32)
    for s in range(NSPLIT):
        g2 = _sc_gather(
            v2e_w, lax.dynamic_slice_in_dim(idx_flat, s * BLS, BLS)
        ).reshape(BLS2, 2 * D)
        r16s = lax.dynamic_slice_in_dim(r16, s * BLS2, BLS2, axis=1)
        acc = _tc_mlp(g2, r16s, acc, r2e_pad, w1a2, w1bT, b1r, w2d, b2d,
                      do_scale=(s == NSPLIT - 1))
    return acc.reshape(B, D)
